# baseline (speedup 1.0000x reference)
import jax
import jax.numpy as jnp
from jax import lax
from jax.experimental import pallas as pl
from jax.experimental.pallas import tpu as pltpu
from jax.experimental.pallas import tpu_sc as plsc

B = 16384
D = 768
CHUNK = 64
LANES = 16

_info = plsc.get_sparse_core_info()
NC, NS = _info.num_cores, _info.num_subcores
NW = NC * NS
B_PER_W = B // NW
N_CHUNKS = B_PER_W // CHUNK


def _lookup_body(idx_hbm, table_hbm, out_hbm, rep_hbm, idx_v, table_v,
                 rows0, rows1, gsem0, gsem1, ssem0, ssem1, rsem):
    wid = lax.axis_index("s") * NC + lax.axis_index("c")
    base = wid * B_PER_W
    bufs = (rows0, rows1)
    gsems = (gsem0, gsem1)
    ssems = (ssem0, ssem1)

    pltpu.sync_copy(table_hbm, table_v)
    rep_copy = pltpu.make_async_copy(
        table_v, rep_hbm.at[pl.ds(2 * wid, 2)], rsem)
    rep_copy.start()

    pltpu.sync_copy(idx_hbm.at[pl.ds(base, B_PER_W)], idx_v)
    off = 2 * wid
    for j in range(B_PER_W // LANES):
        sl = pl.ds(j * LANES, LANES)
        idx_v[sl] = idx_v[sl] + off

    rep_copy.wait()

    def gather(c, b):
        idx_slice = idx_v.at[pl.ds(c * CHUNK, CHUNK)]
        return pltpu.make_async_copy(rep_hbm.at[idx_slice], bufs[b], gsems[b])

    def scatter(c, b):
        dst = out_hbm.at[pl.ds(base + c * CHUNK, CHUNK)]
        return pltpu.make_async_copy(bufs[b], dst, ssems[b])

    def pair(t):
        for b in range(2):
            c = 2 * t + b

            @pl.when(t > 0)
            def _wait_prev_scatter():
                scatter(c, b).wait()

            gather(c, b).start()
            gather(c, b).wait()
            scatter(c, b).start()

    pl.loop(0, N_CHUNKS // 2)(pair)
    for b in range(2):
        scatter(0, b).wait()


@jax.jit
def kernel(input, table):
    mesh = plsc.VectorSubcoreMesh(core_axis_name="c", subcore_axis_name="s")
    run = pl.kernel(
        _lookup_body,
        out_type=(
            jax.ShapeDtypeStruct((B, D), jnp.float32),
            jax.ShapeDtypeStruct((2 * NW, D), jnp.float32),
        ),
        mesh=mesh,
        scratch_types=[
            pltpu.VMEM((B_PER_W,), jnp.int32),
            pltpu.VMEM((2, D), jnp.float32),
            pltpu.VMEM((CHUNK, D), jnp.float32),
            pltpu.VMEM((CHUNK, D), jnp.float32),
            pltpu.SemaphoreType.DMA,
            pltpu.SemaphoreType.DMA,
            pltpu.SemaphoreType.DMA,
            pltpu.SemaphoreType.DMA,
            pltpu.SemaphoreType.DMA,
        ],
    )
    out, _ = run(input, table)
    return out


# speedup vs baseline: 1.6085x; 1.6085x over previous
import jax
import jax.numpy as jnp
from jax import lax
from jax.experimental import pallas as pl
from jax.experimental.pallas import tpu as pltpu
from jax.experimental.pallas import tpu_sc as plsc

B = 16384
D = 768
CHUNK = 64
LANES = 16
SLICES = D // LANES
GROUPS = CHUNK // LANES

_info = plsc.get_sparse_core_info()
NC, NS = _info.num_cores, _info.num_subcores
NW = NC * NS
B_PER_W = B // NW
N_CHUNKS = B_PER_W // CHUNK


def _lookup_body(idx_hbm, table_hbm, out_hbm, idx_v, table_v, rows0, rows1,
                 ssem0, ssem1):
    wid = lax.axis_index("s") * NC + lax.axis_index("c")
    base = wid * B_PER_W
    bufs = (rows0, rows1)
    sems = (ssem0, ssem1)
    pltpu.sync_copy(table_hbm, table_v)
    pltpu.sync_copy(idx_hbm.at[pl.ds(base, B_PER_W)], idx_v)

    def fill_chunk(c, buf):
        CG = 8
        for cg in range(SLICES // CG):
            t0 = [table_v[0, pl.ds((cg * CG + k) * LANES, LANES)]
                  for k in range(CG)]
            dt = [table_v[1, pl.ds((cg * CG + k) * LANES, LANES)] - t0[k]
                  for k in range(CG)]

            def group(g):
                iv = idx_v[pl.ds(c * CHUNK + g * LANES, LANES)]
                fv = iv.astype(jnp.float32)
                for r in range(LANES):
                    fb = lax.gather(
                        fv, jnp.full((LANES, 1), r, jnp.int32),
                        lax.GatherDimensionNumbers(
                            offset_dims=(), collapsed_slice_dims=(0,),
                            start_index_map=(0,)),
                        slice_sizes=(1,),
                        mode=lax.GatherScatterMode.PROMISE_IN_BOUNDS)
                    row = g * LANES + r
                    for k in range(CG):
                        buf[row, pl.ds((cg * CG + k) * LANES, LANES)] = (
                            t0[k] + fb * dt[k])
            pl.loop(0, GROUPS)(group)

    def scatter(c, b):
        dst = out_hbm.at[pl.ds(base + c * CHUNK, CHUNK)]
        return pltpu.make_async_copy(bufs[b], dst, sems[b])

    DIAG_FILL_ONLY = False

    def pair(t):
        for b in range(2):
            c = 2 * t + b

            if not DIAG_FILL_ONLY:
                @pl.when(t > 0)
                def _wait_prev():
                    scatter(c, b).wait()

            fill_chunk(c, bufs[b])
            if not DIAG_FILL_ONLY:
                scatter(c, b).start()

    pl.loop(0, N_CHUNKS // 2)(pair)
    if not DIAG_FILL_ONLY:
        for b in range(2):
            scatter(0, b).wait()


@jax.jit
def kernel(input, table):
    mesh = plsc.VectorSubcoreMesh(core_axis_name="c", subcore_axis_name="s")
    run = pl.kernel(
        _lookup_body,
        out_type=jax.ShapeDtypeStruct((B, D), jnp.float32),
        mesh=mesh,
        scratch_types=[
            pltpu.VMEM((B_PER_W,), jnp.int32),
            pltpu.VMEM((2, D), jnp.float32),
            pltpu.VMEM((CHUNK, D), jnp.float32),
            pltpu.VMEM((CHUNK, D), jnp.float32),
            pltpu.SemaphoreType.DMA,
            pltpu.SemaphoreType.DMA,
        ],
    )
    return run(input, table)


# speedup vs baseline: 1.9877x; 1.2358x over previous
import jax
import jax.numpy as jnp
from jax import lax
from jax.experimental import pallas as pl
from jax.experimental.pallas import tpu as pltpu
from jax.experimental.pallas import tpu_sc as plsc

B = 16384
D = 768
CHUNK = 64
LANES = 16
SLICES = D // LANES
GROUPS = CHUNK // LANES

_info = plsc.get_sparse_core_info()
NC, NS = _info.num_cores, _info.num_subcores
NW = NC * NS
B_PER_W = B // NW
N_CHUNKS = B_PER_W // CHUNK


def _lookup_body(idx_hbm, table_hbm, out_hbm, idx_v, table_v, rows0, rows1,
                 ssem0, ssem1):
    wid = lax.axis_index("s") * NC + lax.axis_index("c")
    base = wid * B_PER_W
    bufs = (rows0, rows1)
    sems = (ssem0, ssem1)
    pltpu.sync_copy(table_hbm, table_v)
    pltpu.sync_copy(idx_hbm.at[pl.ds(base, B_PER_W)], idx_v)

    def fill_chunk(c, buf):
        CG = 8
        for cg in range(SLICES // CG):
            t0 = [table_v[0, pl.ds((cg * CG + k) * LANES, LANES)]
                  for k in range(CG)]
            dt = [table_v[1, pl.ds((cg * CG + k) * LANES, LANES)] - t0[k]
                  for k in range(CG)]

            def group(g):
                iv = idx_v[pl.ds(c * CHUNK + g * LANES, LANES)]
                fv = iv.astype(jnp.float32)
                for r in range(LANES):
                    fb = lax.gather(
                        fv, jnp.full((LANES, 1), r, jnp.int32),
                        lax.GatherDimensionNumbers(
                            offset_dims=(), collapsed_slice_dims=(0,),
                            start_index_map=(0,)),
                        slice_sizes=(1,),
                        mode=lax.GatherScatterMode.PROMISE_IN_BOUNDS)
                    row = g * LANES + r
                    for k in range(CG):
                        buf[row, pl.ds((cg * CG + k) * LANES, LANES)] = (
                            t0[k] + fb * dt[k])
            plsc.parallel_loop(0, GROUPS, unroll=2)(group)

    def scatter(c, b):
        dst = out_hbm.at[pl.ds(base + c * CHUNK, CHUNK)]
        return pltpu.make_async_copy(bufs[b], dst, sems[b])

    DIAG_FILL_ONLY = False

    def pair(t):
        for b in range(2):
            c = 2 * t + b

            if not DIAG_FILL_ONLY:
                @pl.when(t > 0)
                def _wait_prev():
                    scatter(c, b).wait()

            fill_chunk(c, bufs[b])
            if not DIAG_FILL_ONLY:
                scatter(c, b).start()

    pl.loop(0, N_CHUNKS // 2)(pair)
    if not DIAG_FILL_ONLY:
        for b in range(2):
            scatter(0, b).wait()


@jax.jit
def kernel(input, table):
    mesh = plsc.VectorSubcoreMesh(core_axis_name="c", subcore_axis_name="s")
    run = pl.kernel(
        _lookup_body,
        out_type=jax.ShapeDtypeStruct((B, D), jnp.float32),
        mesh=mesh,
        scratch_types=[
            pltpu.VMEM((B_PER_W,), jnp.int32),
            pltpu.VMEM((2, D), jnp.float32),
            pltpu.VMEM((CHUNK, D), jnp.float32),
            pltpu.VMEM((CHUNK, D), jnp.float32),
            pltpu.SemaphoreType.DMA,
            pltpu.SemaphoreType.DMA,
        ],
    )
    return run(input, table)
